# baseline (speedup 1.0000x reference)
import functools

import jax
import jax.numpy as jnp
from jax.experimental import pallas as pl
from jax.experimental.pallas import tpu as pltpu


def _conv_body(x_ref, w_ref, b_ref, o_ref, *, TH, W, C_in, LOAD, NT):
    TS = TH * W
    col = jax.lax.broadcasted_iota(jnp.int32, (C_in, TS), 1) % W
    b = b_ref[...]
    w = w_ref[...]
    for t in range(NT):
        win = x_ref[0, :, t * TS:t * TS + LOAD]
        taps = []
        for kh in range(3):
            for kw in range(3):
                off = (W * 2 - W - 1) + kh * W + kw
                s = win[:, off:off + TS]
                if kw == 0:
                    s = jnp.where(col == 0, 0, s)
                elif kw == 2:
                    s = jnp.where(col == W - 1, 0, s)
                taps.append(s)
        xmat = jnp.concatenate(taps, axis=0)
        acc = jnp.dot(w, xmat, preferred_element_type=jnp.float32)
        o_ref[0, :, t * TS:(t + 1) * TS] = acc + b


def kernel(x, weight, bias):
    N, C_in, H, W = x.shape
    C_out = weight.shape[0]
    TH = 8
    HW = H * W
    TS = TH * W
    front = 2 * W
    LOAD = ((TS + 2 * W + 2 + 127) // 128) * 128
    flat = (H // TH - 1) * TS + LOAD
    back = flat - front - HW

    xf = jnp.pad(x.reshape(N, C_in, HW).astype(jnp.bfloat16),
                 ((0, 0), (0, 0), (front, back)))
    w_mat = weight.transpose(0, 2, 3, 1).reshape(C_out, 9 * C_in)
    w_mat = w_mat.astype(jnp.bfloat16)
    b_col = bias.astype(jnp.float32).reshape(C_out, 1)

    out = pl.pallas_call(
        functools.partial(_conv_body, TH=TH, W=W, C_in=C_in, LOAD=LOAD,
                          NT=H // TH),
        out_shape=jax.ShapeDtypeStruct((N, C_out, HW), jnp.float32),
        grid=(N,),
        in_specs=[
            pl.BlockSpec((1, C_in, flat), lambda n: (n, 0, 0)),
            pl.BlockSpec((C_out, 9 * C_in), lambda n: (0, 0)),
            pl.BlockSpec((C_out, 1), lambda n: (0, 0)),
        ],
        out_specs=pl.BlockSpec((1, C_out, HW), lambda n: (n, 0, 0)),
        compiler_params=pltpu.CompilerParams(
            dimension_semantics=("parallel",)),
    )(xf, w_mat, b_col)
    return out.reshape(N, C_out, H, W)


# speedup vs baseline: 2.0832x; 2.0832x over previous
import functools

import jax
import jax.numpy as jnp
from jax.experimental import pallas as pl
from jax.experimental.pallas import tpu as pltpu


def _conv_body(x_ref, w_ref, b_ref, o_ref, scr, *, TH, W, C_in, LOAD, NT):
    TS = TH * W
    HW = NT * TS
    FLAT = scr.shape[1]
    front = W + 1
    flat = x_ref[0].astype(jnp.bfloat16).reshape(C_in, HW)
    scr[:, 0:2 * W] = jnp.zeros((C_in, 2 * W), jnp.bfloat16)
    scr[:, FLAT - 4 * W:FLAT] = jnp.zeros((C_in, 4 * W), jnp.bfloat16)
    scr[:, front:front + HW] = flat
    col = jax.lax.broadcasted_iota(jnp.int32, (C_in, TS), 1) % W
    b = b_ref[...]
    w = w_ref[...]
    for t in range(NT):
        win = scr[:, t * TS:t * TS + LOAD]
        taps = []
        for kh in range(3):
            for kw in range(3):
                off = kh * W + kw
                s = win[:, off:off + TS]
                if kw == 0:
                    s = jnp.where(col == 0, 0, s)
                elif kw == 2:
                    s = jnp.where(col == W - 1, 0, s)
                taps.append(s)
        xmat = jnp.concatenate(taps, axis=0)
        acc = jnp.dot(w, xmat, preferred_element_type=jnp.float32)
        o_ref[0, :, t * TS:(t + 1) * TS] = acc + b


def kernel(x, weight, bias):
    N, C_in, H, W = x.shape
    C_out = weight.shape[0]
    TH = 8
    HW = H * W
    TS = TH * W
    front = W + 1
    LOAD = ((TS + 2 * W + 2 + 127) // 128) * 128
    flat = (H // TH - 1) * TS + LOAD
    back = flat - front - HW

    w_mat = weight.transpose(0, 2, 3, 1).reshape(C_out, 9 * C_in)
    w_mat = w_mat.astype(jnp.bfloat16)
    b_col = bias.astype(jnp.float32).reshape(C_out, 1)

    out = pl.pallas_call(
        functools.partial(_conv_body, TH=TH, W=W, C_in=C_in, LOAD=LOAD,
                          NT=H // TH),
        out_shape=jax.ShapeDtypeStruct((N, C_out, HW), jnp.float32),
        grid=(N,),
        in_specs=[
            pl.BlockSpec((1, C_in, H, W), lambda n: (n, 0, 0, 0)),
            pl.BlockSpec((C_out, 9 * C_in), lambda n: (0, 0)),
            pl.BlockSpec((C_out, 1), lambda n: (0, 0)),
        ],
        out_specs=pl.BlockSpec((1, C_out, HW), lambda n: (n, 0, 0)),
        scratch_shapes=[pltpu.VMEM((C_in, flat), jnp.bfloat16)],
        compiler_params=pltpu.CompilerParams(
            dimension_semantics=("parallel",)),
    )(x, w_mat, b_col)
    return out.reshape(N, C_out, H, W)
